# baseline (speedup 1.0000x reference)
import functools

import jax
import jax.numpy as jnp
from jax import lax
from jax.experimental import pallas as pl
from jax.experimental.pallas import tpu as pltpu
from jax.experimental.pallas import tpu_sc as plsc

BATCH = 16384
USER_D = 32
ITEM_D = 10
TEST_D = 10
TAG_D = 12

NUM_CORES = 2
NUM_SUBCORES = 16
NUM_WORKERS = NUM_CORES * NUM_SUBCORES
B_PER_W = BATCH // NUM_WORKERS
CHUNK = 128
GROUPS = B_PER_W // 16


def _body(u_idx, i_idx, te_idx, ta_idx, user_W, item_W, test_W, tag_W,
          out_hbm, u_idx_v, i_idx_v, te_idx_v, ta_idx_v,
          u_rows, i_rows, te_rows, ta_rows, out_v,
          sem_u, sem_i, sem_te, sem_ta):
    wid = lax.axis_index("s") * NUM_CORES + lax.axis_index("c")
    base = wid * B_PER_W

    pltpu.sync_copy(u_idx.at[pl.ds(base, B_PER_W)], u_idx_v)
    pltpu.sync_copy(i_idx.at[pl.ds(base, B_PER_W)], i_idx_v)
    pltpu.sync_copy(te_idx.at[pl.ds(base, B_PER_W)], te_idx_v)
    pltpu.sync_copy(ta_idx.at[pl.ds(base, B_PER_W)], ta_idx_v)

    def chunk_body(ch, _):
        ch0 = ch * CHUNK

        @plsc.parallel_loop(0, CHUNK // 16)
        def fetch_body(g):
            g16 = ch0 + g * 16
            uvec = u_idx_v[pl.ds(g16, 16)]
            ivec = i_idx_v[pl.ds(g16, 16)]
            tevec = te_idx_v[pl.ds(g16, 16)]
            tavec = ta_idx_v[pl.ds(g16, 16)]
            for l in range(16):
                rr = g * 16 + l
                pltpu.make_async_copy(user_W.at[pl.ds(uvec[l], 1), :],
                                      u_rows.at[pl.ds(rr, 1), :],
                                      sem_u).start()
                pltpu.make_async_copy(item_W.at[pl.ds(ivec[l], 1), :],
                                      i_rows.at[pl.ds(rr, 1), :],
                                      sem_i).start()
                pltpu.make_async_copy(test_W.at[pl.ds(tevec[l], 1), :],
                                      te_rows.at[pl.ds(rr, 1), :],
                                      sem_te).start()
                pltpu.make_async_copy(tag_W.at[pl.ds(tavec[l], 1), :],
                                      ta_rows.at[pl.ds(rr, 1), :],
                                      sem_ta).start()

        pltpu.make_async_copy(user_W.at[pl.ds(0, CHUNK), :], u_rows,
                              sem_u).wait()
        pltpu.make_async_copy(item_W.at[pl.ds(0, CHUNK), :], i_rows,
                              sem_i).wait()
        pltpu.make_async_copy(test_W.at[pl.ds(0, CHUNK), :], te_rows,
                              sem_te).wait()
        pltpu.make_async_copy(tag_W.at[pl.ds(0, CHUNK), :], ta_rows,
                              sem_ta).wait()

        def group_body(g, _):
            rows = g * 16 + lax.broadcasted_iota(jnp.int32, (16,), 0)
            acc = jnp.zeros((16,), jnp.float32)
            for d in range(USER_D):
                dvec = jnp.full((16,), d, jnp.int32)
                u = plsc.load_gather(u_rows, [rows, dvec])
                if d < ITEM_D:
                    c = plsc.load_gather(i_rows, [rows, dvec])
                elif d < ITEM_D + TEST_D:
                    c = plsc.load_gather(
                        te_rows,
                        [rows, jnp.full((16,), d - ITEM_D, jnp.int32)])
                else:
                    c = plsc.load_gather(
                        ta_rows,
                        [rows, jnp.full((16,), d - ITEM_D - TEST_D,
                                        jnp.int32)])
                acc = acc + u * c
            res = 1.0 / (1.0 + jnp.exp(-acc))
            out_v[pl.ds(ch0 + g * 16, 16)] = res
            return ()

        lax.fori_loop(0, CHUNK // 16, group_body, (), unroll=False)
        return ()

    lax.fori_loop(0, B_PER_W // CHUNK, chunk_body, (), unroll=False)

    pltpu.sync_copy(out_v, out_hbm.at[pl.ds(base, B_PER_W)])


@functools.partial(jax.jit, static_argnames=("interpret",))
def _run(u_idx, i_idx, te_idx, ta_idx, user_W, item_W, test_W, tag_W,
         interpret=False):
    mesh = plsc.VectorSubcoreMesh(core_axis_name="c", subcore_axis_name="s",
                                  num_cores=NUM_CORES,
                                  num_subcores=NUM_SUBCORES)
    idx1 = pltpu.VMEM((B_PER_W,), jnp.int32)
    return pl.kernel(
        _body,
        out_type=jax.ShapeDtypeStruct((BATCH,), jnp.float32),
        mesh=mesh,
        scratch_types=[
            idx1, idx1, idx1, idx1,
            pltpu.VMEM((CHUNK, USER_D), jnp.float32),
            pltpu.VMEM((CHUNK, ITEM_D), jnp.float32),
            pltpu.VMEM((CHUNK, TEST_D), jnp.float32),
            pltpu.VMEM((CHUNK, TAG_D), jnp.float32),
            pltpu.VMEM((B_PER_W,), jnp.float32),
            pltpu.SemaphoreType.DMA,
            pltpu.SemaphoreType.DMA,
            pltpu.SemaphoreType.DMA,
            pltpu.SemaphoreType.DMA,
        ],
        compiler_params=pltpu.CompilerParams(
            use_tc_tiling_on_sc=True,
            needs_layout_passes=False,
        ),
        interpret=interpret,
    )(u_idx, i_idx, te_idx, ta_idx, user_W, item_W, test_W, tag_W)


def kernel(data, user_W, item_W, test_W, tag_W):
    u_idx = data[:, 0]
    i_idx = data[:, 1]
    te_idx = data[:, 2]
    ta_idx = data[:, 3]
    return _run(u_idx, i_idx, te_idx, ta_idx, user_W, item_W, test_W, tag_W)
